# Initial kernel scaffold; baseline (speedup 1.0000x reference)
import jax
import jax.numpy as jnp
from jax.experimental import pallas as pl


def kernel(features, edge_index, W1, b1, W2, b2):
    raise NotImplementedError("write your pallas kernel here")


# speedup vs baseline: 7.9259x; 7.9259x over previous
import functools

import jax
import jax.numpy as jnp
from jax import lax
from jax.experimental import pallas as pl
from jax.experimental.pallas import tpu as pltpu
from jax.experimental.pallas import tpu_sc as plsc

_NC = 2
_NS = 16
_NW = _NC * _NS

_B = 80

_F32 = jnp.float32


def _mesh():
    return plsc.VectorSubcoreMesh(core_axis_name="c", subcore_axis_name="s")


def _zero_tmp(tmp_v, rows, lane_chunks):
    def zrow(r, carry):
        for j in range(lane_chunks):
            tmp_v[r, pl.ds(j * 16, 16)] = jnp.zeros((16,), _F32)
        return carry
    lax.fori_loop(0, rows, zrow, 0)


def _deg_body(n_pad, npw, bpw, srcs3, dsts3, out, srcs_v, dsts_v, ones_v,
              zz_v, wb_v, dega, degb):
    cid = lax.axis_index("c")
    sid = lax.axis_index("s")
    wid = cid * _NS + sid
    for j in range(_B // 16):
        ones_v[pl.ds(j * 16, 16)] = jnp.full((16,), 1.0, _F32)
    for j in range(npw // 16):
        zz_v[pl.ds(j * 16, 16)] = jnp.zeros((16,), _F32)
    pltpu.sync_copy(srcs3.at[wid], srcs_v)
    pltpu.sync_copy(dsts3.at[wid], dsts_v)
    zoff = pl.multiple_of(sid * npw, 128)
    pltpu.sync_copy(zz_v, dega.at[pl.ds(zoff, npw)])
    pltpu.sync_copy(zz_v, degb.at[pl.ds(zoff, npw)])
    plsc.subcore_barrier()

    def ebody(b, carry):
        pltpu.sync_copy(ones_v, dega.at[srcs_v.at[b]], add=True)
        pltpu.sync_copy(ones_v, degb.at[dsts_v.at[b]], add=True)
        return carry
    lax.fori_loop(0, bpw, ebody, 0)
    plsc.subcore_barrier()
    o_a = pl.multiple_of((cid * 2 + 0) * n_pad + sid * npw, 128)
    o_b = pl.multiple_of((cid * 2 + 1) * n_pad + sid * npw, 128)
    pltpu.sync_copy(dega.at[pl.ds(zoff, npw)], wb_v)
    pltpu.sync_copy(wb_v, out.at[pl.ds(o_a, npw)])
    pltpu.sync_copy(degb.at[pl.ds(zoff, npw)], wb_v)
    pltpu.sync_copy(wb_v, out.at[pl.ds(o_b, npw)])


def _deg_call(srcs3, dsts3, n_pad, bpw):
    npw = n_pad // _NS
    body = functools.partial(_deg_body, n_pad, npw, bpw)
    return pl.kernel(
        body,
        out_type=jax.ShapeDtypeStruct((4 * n_pad,), _F32),
        mesh=_mesh(),
        scratch_types=[
            pltpu.VMEM((bpw, _B), jnp.int32),
            pltpu.VMEM((bpw, _B), jnp.int32),
            pltpu.VMEM((_B,), _F32),
            pltpu.VMEM((npw,), _F32),
            pltpu.VMEM((npw,), _F32),
            pltpu.VMEM_SHARED((n_pad,), _F32),
            pltpu.VMEM_SHARED((n_pad,), _F32),
        ],
    )(srcs3, dsts3)


def _spmm_body(n_pad, bpw, n_chunks, *refs):
    xn_refs = refs[:n_chunks]
    srcs3, dsts3, out = refs[n_chunks:n_chunks + 3]
    srcs_v, dsts_v, rows_v, acc, sem = refs[n_chunks + 3:]
    cid = lax.axis_index("c")
    sid = lax.axis_index("s")
    wid = cid * _NS + sid
    rps = n_pad // _NS
    wrows = _B
    nchk = rps // wrows

    pltpu.sync_copy(srcs3.at[wid], srcs_v)
    pltpu.sync_copy(dsts3.at[wid], dsts_v)

    for c in range(n_chunks):
        xn_c = xn_refs[c]
        _zero_tmp(rows_v, wrows, 8)
        for i in range(nchk):
            r0 = pl.multiple_of(sid * rps + i * wrows, 16)
            pltpu.sync_copy(rows_v, acc.at[pl.ds(r0, wrows), :])
        plsc.subcore_barrier()

        def ebody(b, carry):
            pltpu.async_copy(xn_c.at[srcs_v.at[b]], rows_v, sem).wait()
            pltpu.sync_copy(rows_v, acc.at[dsts_v.at[b]], add=True)
            return carry
        lax.fori_loop(0, bpw, ebody, 0)
        plsc.subcore_barrier()
        for i in range(nchk):
            r0 = pl.multiple_of(sid * rps + i * wrows, 16)
            pltpu.sync_copy(acc.at[pl.ds(r0, wrows), :], rows_v)
            pltpu.sync_copy(rows_v, out.at[cid, c, pl.ds(r0, wrows), :])


def _spmm_call(xn_list, srcs3, dsts3, n_pad, bpw):
    n_chunks = len(xn_list)
    body = functools.partial(_spmm_body, n_pad, bpw, n_chunks)
    return pl.kernel(
        body,
        out_type=jax.ShapeDtypeStruct((_NC, n_chunks, n_pad, 128), _F32),
        mesh=_mesh(),
        scratch_types=[
            pltpu.VMEM((bpw, _B), jnp.int32),
            pltpu.VMEM((bpw, _B), jnp.int32),
            pltpu.VMEM((_B, 128), _F32),
            pltpu.VMEM_SHARED((n_pad, 128), _F32),
            pltpu.SemaphoreType.DMA,
        ],
    )(*xn_list, srcs3, dsts3)


def _norm_body(deg4_ref, x_ref, xn_ref, io_ref, ii_ref):
    d = deg4_ref[...]
    dout = d[:, 0:1] + d[:, 2:3]
    din = d[:, 1:2] + d[:, 3:4]
    io = lax.rsqrt(jnp.maximum(dout, 1.0))
    ii = lax.rsqrt(jnp.maximum(din, 1.0))
    xn_ref[...] = x_ref[...] * io
    io_ref[...] = io
    ii_ref[...] = ii


def _norm_call(deg4, x, n, rb):
    grid = (n // rb,)
    return pl.pallas_call(
        _norm_body,
        grid=grid,
        in_specs=[
            pl.BlockSpec((rb, 4), lambda i: (i, 0)),
            pl.BlockSpec((rb, 128), lambda i: (i, 0)),
        ],
        out_specs=[
            pl.BlockSpec((rb, 128), lambda i: (i, 0)),
            pl.BlockSpec((rb, 1), lambda i: (i, 0)),
            pl.BlockSpec((rb, 1), lambda i: (i, 0)),
        ],
        out_shape=[
            jax.ShapeDtypeStruct((n, 128), _F32),
            jax.ShapeDtypeStruct((n, 1), _F32),
            jax.ShapeDtypeStruct((n, 1), _F32),
        ],
    )(deg4, x)


def _l1_body(u2_ref, w1_ref, b1_ref, ii_ref, io_ref, zn_ref):
    um = u2_ref[0] + u2_ref[1]
    y = jnp.dot(um, w1_ref[...], preferred_element_type=_F32,
                precision=lax.Precision.HIGHEST)
    y = y * ii_ref[...] + b1_ref[...]
    zn_ref[0] = jnp.maximum(y, 0.0) * io_ref[...]


def _l1_call(u2, w1, b1r, ii, io, n, rb):
    grid = (4, n // rb)
    return pl.pallas_call(
        _l1_body,
        grid=grid,
        in_specs=[
            pl.BlockSpec((2, rb, 128), lambda k, i: (0, i, 0)),
            pl.BlockSpec((128, 128), lambda k, i: (0, k)),
            pl.BlockSpec((1, 128), lambda k, i: (0, k)),
            pl.BlockSpec((rb, 1), lambda k, i: (i, 0)),
            pl.BlockSpec((rb, 1), lambda k, i: (i, 0)),
        ],
        out_specs=pl.BlockSpec((1, rb, 128), lambda k, i: (k, i, 0)),
        out_shape=jax.ShapeDtypeStruct((4, n, 128), _F32),
    )(u2, w1, b1r, ii, io)


def _l2_body(v2_ref, w2_ref, b2_ref, ii_ref, out_ref):
    acc = jnp.zeros(out_ref.shape, out_ref.dtype)
    for k in range(4):
        vk = v2_ref[0, k] + v2_ref[1, k]
        acc = acc + jnp.dot(vk, w2_ref[k], preferred_element_type=_F32,
                            precision=lax.Precision.HIGHEST)
    out_ref[...] = acc * ii_ref[...] + b2_ref[...]


def _l2_call(v2, w2r, b2r, ii, n, rb):
    grid = (n // rb,)
    return pl.pallas_call(
        _l2_body,
        grid=grid,
        in_specs=[
            pl.BlockSpec((2, 4, rb, 128), lambda i: (0, 0, i, 0)),
            pl.BlockSpec((4, 128, 512), lambda i: (0, 0, 0)),
            pl.BlockSpec((1, 512), lambda i: (0, 0)),
            pl.BlockSpec((rb, 1), lambda i: (i, 0)),
        ],
        out_specs=pl.BlockSpec((rb, 512), lambda i: (i, 0)),
        out_shape=jax.ShapeDtypeStruct((n, 512), _F32),
    )(v2, w2r, b2r, ii)


def kernel(features, edge_index, W1, b1, W2, b2):
    n, d_in = features.shape
    e = edge_index.shape[1]
    d_h = W1.shape[1]
    assert d_in == 128 and d_h == 512
    assert e % (_NW * _B) == 0
    bpw = e // (_NW * _B)
    n_pad = ((n + _NS * 128 - 1) // (_NS * 128)) * (_NS * 128)
    rb = 400
    assert n % rb == 0

    src = edge_index[0]
    dst = edge_index[1]
    srcs3 = src.reshape(_NW, bpw, _B)
    dsts3 = dst.reshape(_NW, bpw, _B)

    deg = _deg_call(srcs3, dsts3, n_pad, bpw)
    deg4 = jnp.transpose(deg.reshape(4, n_pad)[:, :n])

    xn, io, ii = _norm_call(deg4, features, n, rb)

    u2 = _spmm_call([xn], srcs3, dsts3, n_pad, bpw)
    u2 = u2[:, 0, :n, :]

    zn = _l1_call(u2, W1, b1.reshape(1, d_h), ii, io, n, rb)

    v2 = _spmm_call([zn[0], zn[1], zn[2], zn[3]], srcs3, dsts3, n_pad, bpw)
    v2 = v2[:, :, :n, :]

    out = _l2_call(v2, W2.reshape(4, 128, d_h), b2.reshape(1, d_h), ii, n, rb)
    return out


# speedup vs baseline: 11.8785x; 1.4987x over previous
import functools

import jax
import jax.numpy as jnp
from jax import lax
from jax.experimental import pallas as pl
from jax.experimental.pallas import tpu as pltpu
from jax.experimental.pallas import tpu_sc as plsc

_NC = 2
_NS = 16
_NW = _NC * _NS

_B = 80
_WR = 80

_F32 = jnp.float32


def _mesh():
    return plsc.VectorSubcoreMesh(core_axis_name="c", subcore_axis_name="s")


def _zero_tmp(tmp_v, rows, lane_chunks):
    def zrow(r, carry):
        for j in range(lane_chunks):
            tmp_v[r, pl.ds(j * 16, 16)] = jnp.zeros((16,), _F32)
        return carry
    lax.fori_loop(0, rows, zrow, 0)


def _deg_body(n_pad, npw, bpw, srcs3, dsts3, out, srcs_v, dsts_v, ones_v,
              zz_v, wb_v, dega, degb):
    cid = lax.axis_index("c")
    sid = lax.axis_index("s")
    wid = cid * _NS + sid
    for j in range(112 // 16):
        ones_v[pl.ds(j * 16, 16)] = jnp.full((16,), 1.0, _F32)
    for j in range(npw // 16):
        zz_v[pl.ds(j * 16, 16)] = jnp.zeros((16,), _F32)
    pltpu.sync_copy(srcs3.at[wid], srcs_v)
    pltpu.sync_copy(dsts3.at[wid], dsts_v)
    zoff = pl.multiple_of(sid * npw, 128)
    pltpu.sync_copy(zz_v, dega.at[pl.ds(zoff, npw)])
    pltpu.sync_copy(zz_v, degb.at[pl.ds(zoff, npw)])
    plsc.subcore_barrier()

    def ebody(b, carry):
        pltpu.sync_copy(ones_v.at[pl.ds(0, _B)], dega.at[srcs_v.at[b]], add=True)
        pltpu.sync_copy(ones_v.at[pl.ds(0, _B)], degb.at[dsts_v.at[b]], add=True)
        return carry
    lax.fori_loop(0, bpw, ebody, 0)
    plsc.subcore_barrier()
    o_a = pl.multiple_of((cid * 2 + 0) * n_pad + sid * npw, 128)
    o_b = pl.multiple_of((cid * 2 + 1) * n_pad + sid * npw, 128)
    pltpu.sync_copy(dega.at[pl.ds(zoff, npw)], wb_v)
    pltpu.sync_copy(wb_v, out.at[pl.ds(o_a, npw)])
    pltpu.sync_copy(degb.at[pl.ds(zoff, npw)], wb_v)
    pltpu.sync_copy(wb_v, out.at[pl.ds(o_b, npw)])


def _deg_call(srcs3, dsts3, n_pad, bpw):
    npw = n_pad // _NS
    body = functools.partial(_deg_body, n_pad, npw, bpw)
    return pl.kernel(
        body,
        out_type=jax.ShapeDtypeStruct((4 * n_pad,), _F32),
        mesh=_mesh(),
        scratch_types=[
            pltpu.VMEM((bpw, _B), jnp.int32),
            pltpu.VMEM((bpw, _B), jnp.int32),
            pltpu.VMEM((112,), _F32),
            pltpu.VMEM((npw,), _F32),
            pltpu.VMEM((npw,), _F32),
            pltpu.VMEM_SHARED((n_pad,), _F32),
            pltpu.VMEM_SHARED((n_pad,), _F32),
        ],
    )(srcs3, dsts3)


def _spmm_body(n_pad, bpw, n_chunks, *refs):
    xn_refs = refs[:n_chunks]
    pk3, out = refs[n_chunks:n_chunks + 2]
    pk_v, sa, sb, dv, ra, rb_, acc, s0, s1 = refs[n_chunks + 2:]
    cid = lax.axis_index("c")
    sid = lax.axis_index("s")
    wid = cid * _NS + sid
    rps = n_pad // _NS
    nchk = rps // _WR

    def _unp_src(b, sref):
        for j in range(_B // 16):
            p = pk_v[b, pl.ds(j * 16, 16)]
            sref[pl.ds(j * 16, 16)] = lax.bitwise_and(p, 16383)

    def _unp_dst(b):
        for j in range(_B // 16):
            p = pk_v[b, pl.ds(j * 16, 16)]
            dv[pl.ds(j * 16, 16)] = lax.shift_right_logical(p, 14)

    pltpu.sync_copy(pk3.at[wid], pk_v)
    _zero_tmp(ra, _WR, 8)

    for c in range(n_chunks):
        xn_c = xn_refs[c]
        if c > 0:
            _zero_tmp(ra, _WR, 8)
        for i in range(nchk):
            r0 = pl.multiple_of(sid * rps + i * _WR, 16)
            pltpu.sync_copy(ra.at[pl.ds(0, _WR), :], acc.at[pl.ds(r0, _WR), :])
        plsc.subcore_barrier()

        _unp_src(0, sa)
        pltpu.async_copy(xn_c.at[sa], ra, s0)

        def ebody(t, carry):
            b0 = 2 * t
            _unp_src(b0 + 1, sb)
            pltpu.async_copy(xn_c.at[sb], rb_, s1)
            pltpu.make_async_copy(xn_c.at[sa], ra, s0).wait()
            _unp_dst(b0)
            pltpu.sync_copy(ra, acc.at[dv], add=True)
            _unp_src(b0 + 2, sa)
            pltpu.async_copy(xn_c.at[sa], ra, s0)
            pltpu.make_async_copy(xn_c.at[sb], rb_, s1).wait()
            _unp_dst(b0 + 1)
            pltpu.sync_copy(rb_, acc.at[dv], add=True)
            return carry
        lax.fori_loop(0, (bpw - 1) // 2, ebody, 0)
        pltpu.make_async_copy(xn_c.at[sa], ra, s0).wait()
        _unp_dst(bpw - 1)
        pltpu.sync_copy(ra, acc.at[dv], add=True)

        plsc.subcore_barrier()
        for i in range(nchk):
            r0 = pl.multiple_of(sid * rps + i * _WR, 16)
            pltpu.sync_copy(acc.at[pl.ds(r0, _WR), :], ra.at[pl.ds(0, _WR), :])
            pltpu.sync_copy(ra.at[pl.ds(0, _WR), :],
                            out.at[cid, c, pl.ds(r0, _WR), :])


def _spmm_call(xn_list, pk3, n_pad, bpw):
    n_chunks = len(xn_list)
    body = functools.partial(_spmm_body, n_pad, bpw, n_chunks)
    return pl.kernel(
        body,
        out_type=jax.ShapeDtypeStruct((_NC, n_chunks, n_pad, 128), _F32),
        mesh=_mesh(),
        scratch_types=[
            pltpu.VMEM((bpw, _B), jnp.int32),
            pltpu.VMEM((_B,), jnp.int32),
            pltpu.VMEM((_B,), jnp.int32),
            pltpu.VMEM((_B,), jnp.int32),
            pltpu.VMEM((_B, 128), _F32),
            pltpu.VMEM((_B, 128), _F32),
            pltpu.VMEM_SHARED((n_pad, 128), _F32),
            pltpu.SemaphoreType.DMA,
            pltpu.SemaphoreType.DMA,
        ],
    )(*xn_list, pk3)


def _norm_body(deg4_ref, x_ref, xn_ref, io_ref, ii_ref):
    d = deg4_ref[...]
    dout = d[:, 0:1] + d[:, 2:3]
    din = d[:, 1:2] + d[:, 3:4]
    io = lax.rsqrt(jnp.maximum(dout, 1.0))
    ii = lax.rsqrt(jnp.maximum(din, 1.0))
    xn_ref[...] = x_ref[...] * io
    io_ref[...] = io
    ii_ref[...] = ii


def _norm_call(deg4, x, n, rb):
    grid = (n // rb,)
    return pl.pallas_call(
        _norm_body,
        grid=grid,
        in_specs=[
            pl.BlockSpec((rb, 4), lambda i: (i, 0)),
            pl.BlockSpec((rb, 128), lambda i: (i, 0)),
        ],
        out_specs=[
            pl.BlockSpec((rb, 128), lambda i: (i, 0)),
            pl.BlockSpec((rb, 1), lambda i: (i, 0)),
            pl.BlockSpec((rb, 1), lambda i: (i, 0)),
        ],
        out_shape=[
            jax.ShapeDtypeStruct((n, 128), _F32),
            jax.ShapeDtypeStruct((n, 1), _F32),
            jax.ShapeDtypeStruct((n, 1), _F32),
        ],
    )(deg4, x)


def _l1_body(u2_ref, w1_ref, b1_ref, ii_ref, io_ref, zn_ref):
    um = u2_ref[0] + u2_ref[1]
    y = jnp.dot(um, w1_ref[...], preferred_element_type=_F32,
                precision=lax.Precision.HIGHEST)
    y = y * ii_ref[...] + b1_ref[...]
    zn_ref[0] = jnp.maximum(y, 0.0) * io_ref[...]


def _l1_call(u2, w1, b1r, ii, io, n, rb):
    grid = (4, n // rb)
    return pl.pallas_call(
        _l1_body,
        grid=grid,
        in_specs=[
            pl.BlockSpec((2, rb, 128), lambda k, i: (0, i, 0)),
            pl.BlockSpec((128, 128), lambda k, i: (0, k)),
            pl.BlockSpec((1, 128), lambda k, i: (0, k)),
            pl.BlockSpec((rb, 1), lambda k, i: (i, 0)),
            pl.BlockSpec((rb, 1), lambda k, i: (i, 0)),
        ],
        out_specs=pl.BlockSpec((1, rb, 128), lambda k, i: (k, i, 0)),
        out_shape=jax.ShapeDtypeStruct((4, n, 128), _F32),
    )(u2, w1, b1r, ii, io)


def _l2_body(v2_ref, w2_ref, b2_ref, ii_ref, out_ref):
    acc = jnp.zeros(out_ref.shape, out_ref.dtype)
    for k in range(4):
        vk = v2_ref[0, k] + v2_ref[1, k]
        acc = acc + jnp.dot(vk, w2_ref[k], preferred_element_type=_F32,
                            precision=lax.Precision.HIGHEST)
    out_ref[...] = acc * ii_ref[...] + b2_ref[...]


def _l2_call(v2, w2r, b2r, ii, n, rb):
    grid = (n // rb,)
    return pl.pallas_call(
        _l2_body,
        grid=grid,
        in_specs=[
            pl.BlockSpec((2, 4, rb, 128), lambda i: (0, 0, i, 0)),
            pl.BlockSpec((4, 128, 512), lambda i: (0, 0, 0)),
            pl.BlockSpec((1, 512), lambda i: (0, 0)),
            pl.BlockSpec((rb, 1), lambda i: (i, 0)),
        ],
        out_specs=pl.BlockSpec((rb, 512), lambda i: (i, 0)),
        out_shape=jax.ShapeDtypeStruct((n, 512), _F32),
    )(v2, w2r, b2r, ii)


def kernel(features, edge_index, W1, b1, W2, b2):
    n, d_in = features.shape
    e = edge_index.shape[1]
    d_h = W1.shape[1]
    assert d_in == 128 and d_h == 512
    assert e % (_NW * _B) == 0
    bpw = e // (_NW * _B)
    n_pad = ((n + _NS * 128 - 1) // (_NS * 128)) * (_NS * 128)
    rb = 400
    assert n % rb == 0

    src = edge_index[0]
    dst = edge_index[1]
    srcs3 = src.reshape(_NW, bpw, _B)
    dsts3 = dst.reshape(_NW, bpw, _B)
    pk3 = (src + dst * 16384).reshape(_NW, bpw, _B)

    deg = _deg_call(srcs3, dsts3, n_pad, bpw)
    deg4 = jnp.transpose(deg.reshape(4, n_pad)[:, :n])

    xn, io, ii = _norm_call(deg4, features, n, rb)

    u2 = _spmm_call([xn], pk3, n_pad, bpw)
    u2 = u2[:, 0, :n, :]

    zn = _l1_call(u2, W1, b1.reshape(1, d_h), ii, io, n, rb)

    v2 = _spmm_call([zn[0], zn[1], zn[2], zn[3]], pk3, n_pad, bpw)
    v2 = v2[:, :, :n, :]

    out = _l2_call(v2, W2.reshape(4, 128, d_h), b2.reshape(1, d_h), ii, n, rb)
    return out
